# baseline (speedup 1.0000x reference)
import functools

import jax
import jax.numpy as jnp
from jax import lax
from jax.experimental import pallas as pl
from jax.experimental.pallas import tpu as pltpu
from jax.experimental.pallas import tpu_sc as plsc

_NC = 2
_NS = 16
_NW = _NC * _NS
_ROW = 128


def _ssb_matmul(d, x, meanY):
    P, K = d.shape
    NP = x.shape[1]
    BN = 2048

    def body(d_ref, x_ref, my_ref, o_ref):
        o_ref[...] = (
            lax.dot_general(
                d_ref[...], x_ref[...],
                (((1,), (0,)), ((), ())),
                preferred_element_type=jnp.float32,
            )
            + my_ref[...]
        )

    return pl.pallas_call(
        body,
        grid=(NP // BN,),
        in_specs=[
            pl.BlockSpec((P, K), lambda i: (0, 0)),
            pl.BlockSpec((K, BN), lambda i: (0, i)),
            pl.BlockSpec((1, BN), lambda i: (0, i)),
        ],
        out_specs=pl.BlockSpec((P, BN), lambda i: (0, i)),
        out_shape=jax.ShapeDtypeStruct((P, NP), jnp.float32),
    )(d, x, meanY)


def _sc_scatter(idx2, val2, zeros1d):
    ROWS = idx2.shape[0]
    N = zeros1d.shape[0]
    RPW = ROWS // _NW
    SEG = N // _NS

    mesh = plsc.VectorSubcoreMesh(core_axis_name="c", subcore_axis_name="s")

    @functools.partial(
        pl.kernel,
        out_type=jax.ShapeDtypeStruct((_NC * N,), jnp.float32),
        mesh=mesh,
        scratch_types=[
            pltpu.VMEM((RPW, _ROW), jnp.int32),
            pltpu.VMEM((RPW, _ROW), jnp.float32),
            pltpu.VMEM((SEG,), jnp.float32),
            pltpu.VMEM_SHARED((N,), jnp.float32),
            pltpu.SemaphoreType.DMA,
            pltpu.SemaphoreType.DMA,
        ],
    )
    def sc_body(idx_hbm, val_hbm, z_hbm, out_hbm, idx_v, val_v, stage_v, acc_sh,
                sem, load_sem):
        c = lax.axis_index("c")
        s = lax.axis_index("s")
        wid = s * _NC + c

        idx_load = pltpu.async_copy(
            idx_hbm.at[pl.ds(wid * RPW, RPW)], idx_v, load_sem)
        val_load = pltpu.async_copy(
            val_hbm.at[pl.ds(wid * RPW, RPW)], val_v, load_sem)

        pltpu.sync_copy(z_hbm.at[pl.ds(s * SEG, SEG)], stage_v)
        pltpu.sync_copy(stage_v, acc_sh.at[pl.ds(s * SEG, SEG)])
        plsc.subcore_barrier()

        idx_load.wait()
        val_load.wait()

        def body(j, carry):
            pltpu.async_copy(val_v.at[j], acc_sh.at[idx_v.at[j]], sem,
                             add=True)
            return carry

        lax.fori_loop(0, RPW, body, 0)
        pltpu.make_async_copy(
            val_hbm.at[pl.ds(wid * RPW, RPW)], val_v, sem
        ).wait()
        plsc.subcore_barrier()

        pltpu.sync_copy(acc_sh.at[pl.ds(s * SEG, SEG)], stage_v)
        pltpu.sync_copy(stage_v, out_hbm.at[pl.ds(c * N + s * SEG, SEG)])

    return sc_body(idx2, val2, zeros1d)


def _vprep(partials, dsr, nppr, vbr, sRefr, lam2_11):
    N = dsr.shape[1]

    def body(p_ref, ds_ref, npp_ref, vb_ref, sr_ref, l2_ref, o_ref):
        l2 = l2_ref[0, 0]
        psum = p_ref[0:1, :] + p_ref[1:2, :]
        o_ref[...] = (l2 * ds_ref[...] + psum) / (l2 + npp_ref[...]) \
            * vb_ref[...] + sr_ref[...]

    return pl.pallas_call(
        body,
        out_shape=jax.ShapeDtypeStruct((1, N), jnp.float32),
    )(partials, dsr, nppr, vbr, sRefr, lam2_11)


def _tc_matvec_sq(vrow, A_full, Tarr_top, MT):
    N = A_full.shape[1]
    BN = 4096
    steps = N // BN

    def body(v_ref, a_ref, t_ref, o_ref, acc_ref):
        i = pl.program_id(0)
        part = lax.dot_general(
            a_ref[...], v_ref[...],
            (((1,), (1,)), ((), ())),
            preferred_element_type=jnp.float32,
            precision=lax.Precision.HIGHEST,
        )

        @pl.when(i == 0)
        def _():
            acc_ref[...] = part

        @pl.when(i > 0)
        def _():
            acc_ref[...] = acc_ref[...] + part

        @pl.when(i == steps - 1)
        def _():
            r = acc_ref[...] - t_ref[...]
            o_ref[...] = jnp.sum(r * r).reshape(1, 1)

    return pl.pallas_call(
        body,
        grid=(steps,),
        in_specs=[
            pl.BlockSpec((1, BN), lambda i: (0, i)),
            pl.BlockSpec((MT, BN), lambda i: (0, i)),
            pl.BlockSpec((MT, 1), lambda i: (0, 0)),
        ],
        out_specs=pl.BlockSpec((1, 1), lambda i: (0, 0)),
        out_shape=jax.ShapeDtypeStruct((1, 1), jnp.float32),
        scratch_shapes=[pltpu.VMEM((MT, 1), jnp.float32)],
    )(vrow, A_full, Tarr_top)


def _sc_matvec(a_flat, v1d, mtc, rpt, N):
    CH = 16384
    NCH = N // CH

    mesh = plsc.VectorSubcoreMesh(core_axis_name="c", subcore_axis_name="s")

    @functools.partial(
        pl.kernel,
        out_type=jax.ShapeDtypeStruct((_NW, rpt * 16), jnp.float32),
        mesh=mesh,
        scratch_types=[
            pltpu.VMEM((N,), jnp.float32),
            pltpu.VMEM((CH,), jnp.float32),
            pltpu.VMEM((CH,), jnp.float32),
            pltpu.VMEM((rpt * 16,), jnp.float32),
            pltpu.SemaphoreType.DMA,
            pltpu.SemaphoreType.DMA,
            pltpu.SemaphoreType.DMA,
        ],
    )
    def body(a_hbm, v_hbm, out_hbm, v_v, ab0, ab1, yb, sem0, sem1, vsem):
        c = lax.axis_index("c")
        s = lax.axis_index("s")
        wid = s * _NC + c
        row0 = mtc + wid * rpt

        vload = pltpu.async_copy(v_hbm, v_v, vsem)

        bufs = (ab0, ab1)
        sems = (sem0, sem1)
        total = rpt * NCH

        def start(k):
            row = row0 + (k // NCH)
            off = row * N + (k % NCH) * CH
            return pltpu.async_copy(a_hbm.at[pl.ds(off, CH)],
                                    bufs[k % 2], sems[k % 2])

        pend = start(0)
        vload.wait()

        zero16 = jnp.zeros((16,), jnp.float32)
        accs = [zero16, zero16, zero16, zero16]
        U = 8

        for k in range(total):
            nxt = start(k + 1) if k + 1 < total else None
            pend.wait()
            ab = bufs[k % 2]
            vbase = (k % NCH) * CH

            def mk_body(ab, vbase):
                def inner(j, carry):
                    a0, a1, a2, a3 = carry
                    base = j * (U * 16)
                    cs = [a0, a1, a2, a3]
                    for u in range(U):
                        av = ab[pl.ds(base + u * 16, 16)]
                        vv = v_v[pl.ds(vbase + base + u * 16, 16)]
                        cs[u % 4] = cs[u % 4] + av * vv
                    return tuple(cs)
                return inner

            accs = list(lax.fori_loop(0, CH // (U * 16),
                                      mk_body(ab, vbase),
                                      tuple(accs)))

            if k % NCH == NCH - 1:
                r = k // NCH
                yb[pl.ds(r * 16, 16)] = accs[0] + accs[1] + accs[2] + accs[3]
                accs = [zero16, zero16, zero16, zero16]

            pend = nxt

        pltpu.sync_copy(yb, out_hbm.at[wid])

    return body(a_flat, v1d)


def kernel(d, x, ss, vb, npatches, patches, npp, sRef, A, Tarr, meanY, ds,
           lam2, device):
    P = d.shape[0]
    NP = x.shape[1]
    N = ss.shape[0]
    M = A.shape[0]

    ssb = _ssb_matmul(d, x, meanY)

    idx2 = patches.reshape(P * NP // _ROW, _ROW)
    val2 = ssb.reshape(P * NP // _ROW, _ROW)
    partials = _sc_scatter(idx2, val2, ss.reshape(N))
    partials = partials.reshape(_NC, N)

    vrow = _vprep(
        partials,
        ds.reshape(1, N),
        npp.reshape(1, N),
        vb.reshape(1, N),
        sRef.reshape(1, N),
        lam2.reshape(1, 1),
    )

    RPT = 5
    MSC = _NW * RPT
    MTC = M - MSC

    y_sc = _sc_matvec(A.reshape(M * N), vrow.reshape(N), MTC, RPT, N)
    sq_tc = _tc_matvec_sq(vrow, A, Tarr[:MTC], MTC)

    y_rows = y_sc.reshape(MSC, 16).sum(axis=1)
    sq_sc = jnp.sum((y_rows - Tarr[MTC:, 0]) ** 2)
    return (sq_tc.reshape(()) + sq_sc) / M


# speedup vs baseline: 2.4368x; 2.4368x over previous
import functools

import jax
import jax.numpy as jnp
from jax import lax
from jax.experimental import pallas as pl
from jax.experimental.pallas import tpu as pltpu
from jax.experimental.pallas import tpu_sc as plsc

_NC = 2
_NS = 16
_NW = _NC * _NS
_ROW = 128


def _ssb_matmul(d, x, meanY):
    P, K = d.shape
    NP = x.shape[1]
    BN = 2048

    def body(d_ref, x_ref, my_ref, o_ref):
        o_ref[...] = (
            lax.dot_general(
                d_ref[...], x_ref[...],
                (((1,), (0,)), ((), ())),
                preferred_element_type=jnp.float32,
            )
            + my_ref[...]
        )

    return pl.pallas_call(
        body,
        grid=(NP // BN,),
        in_specs=[
            pl.BlockSpec((P, K), lambda i: (0, 0)),
            pl.BlockSpec((K, BN), lambda i: (0, i)),
            pl.BlockSpec((1, BN), lambda i: (0, i)),
        ],
        out_specs=pl.BlockSpec((P, BN), lambda i: (0, i)),
        out_shape=jax.ShapeDtypeStruct((P, NP), jnp.float32),
    )(d, x, meanY)


def _sc_scatter(idx2, val2, zeros1d):
    ROWS = idx2.shape[0]
    N = zeros1d.shape[0]
    RPW = ROWS // _NW
    SEG = N // _NS

    mesh = plsc.VectorSubcoreMesh(core_axis_name="c", subcore_axis_name="s")

    @functools.partial(
        pl.kernel,
        out_type=jax.ShapeDtypeStruct((_NC * N,), jnp.float32),
        mesh=mesh,
        scratch_types=[
            pltpu.VMEM((RPW, _ROW), jnp.int32),
            pltpu.VMEM((RPW, _ROW), jnp.float32),
            pltpu.VMEM((SEG,), jnp.float32),
            pltpu.VMEM_SHARED((N,), jnp.float32),
            pltpu.SemaphoreType.DMA,
            pltpu.SemaphoreType.DMA,
        ],
    )
    def sc_body(idx_hbm, val_hbm, z_hbm, out_hbm, idx_v, val_v, stage_v, acc_sh,
                sem, load_sem):
        c = lax.axis_index("c")
        s = lax.axis_index("s")
        wid = s * _NC + c

        idx_load = pltpu.async_copy(
            idx_hbm.at[pl.ds(wid * RPW, RPW)], idx_v, load_sem)
        val_load = pltpu.async_copy(
            val_hbm.at[pl.ds(wid * RPW, RPW)], val_v, load_sem)

        pltpu.sync_copy(z_hbm.at[pl.ds(s * SEG, SEG)], stage_v)
        pltpu.sync_copy(stage_v, acc_sh.at[pl.ds(s * SEG, SEG)])
        plsc.subcore_barrier()

        idx_load.wait()
        val_load.wait()

        def body(j, carry):
            pltpu.async_copy(val_v.at[j], acc_sh.at[idx_v.at[j]], sem,
                             add=True)
            return carry

        lax.fori_loop(0, RPW, body, 0)
        pltpu.make_async_copy(
            val_hbm.at[pl.ds(wid * RPW, RPW)], val_v, sem
        ).wait()
        plsc.subcore_barrier()

        pltpu.sync_copy(acc_sh.at[pl.ds(s * SEG, SEG)], stage_v)
        pltpu.sync_copy(stage_v, out_hbm.at[pl.ds(c * N + s * SEG, SEG)])

    return sc_body(idx2, val2, zeros1d)


def _matvec_mse(partials, dsr, nppr, vbr, sRefr, A, Tarr, lam2_11):
    M, N = A.shape
    BN = 4096
    steps = N // BN
    H = M // 2

    def body(p_ref, ds_ref, npp_ref, vb_ref, sr_ref, a0_ref, a1_ref, t_ref,
             l2_ref, o_ref, acc_ref):
        i = pl.program_id(0)
        l2 = l2_ref[0, 0]
        psum = p_ref[0:1, :] + p_ref[1:2, :]
        v = (l2 * ds_ref[...] + psum) / (l2 + npp_ref[...]) * vb_ref[...] \
            + sr_ref[...]
        dims = (((1,), (1,)), ((), ()))
        part = jnp.concatenate(
            [
                lax.dot_general(a0_ref[...], v, dims,
                                preferred_element_type=jnp.float32,
                                precision=lax.Precision.HIGHEST),
                lax.dot_general(a1_ref[...], v, dims,
                                preferred_element_type=jnp.float32,
                                precision=lax.Precision.HIGHEST),
            ],
            axis=0,
        )

        @pl.when(i == 0)
        def _():
            acc_ref[...] = part

        @pl.when(i > 0)
        def _():
            acc_ref[...] = acc_ref[...] + part

        @pl.when(i == steps - 1)
        def _():
            r = acc_ref[...] - t_ref[...]
            o_ref[...] = jnp.sum(r * r).reshape(1, 1) / M

    return pl.pallas_call(
        body,
        grid=(steps,),
        in_specs=[
            pl.BlockSpec((2, BN), lambda i: (0, i)),
            pl.BlockSpec((1, BN), lambda i: (0, i)),
            pl.BlockSpec((1, BN), lambda i: (0, i)),
            pl.BlockSpec((1, BN), lambda i: (0, i)),
            pl.BlockSpec((1, BN), lambda i: (0, i)),
            pl.BlockSpec((H, BN), lambda i: (0, i)),
            pl.BlockSpec((H, BN), lambda i: (1, i)),
            pl.BlockSpec((M, 1), lambda i: (0, 0)),
            pl.BlockSpec((1, 1), lambda i: (0, 0)),
        ],
        out_specs=pl.BlockSpec((1, 1), lambda i: (0, 0)),
        out_shape=jax.ShapeDtypeStruct((1, 1), jnp.float32),
        scratch_shapes=[pltpu.VMEM((M, 1), jnp.float32)],
    )(partials, dsr, nppr, vbr, sRefr, A, A, Tarr, lam2_11)


def kernel(d, x, ss, vb, npatches, patches, npp, sRef, A, Tarr, meanY, ds,
           lam2, device):
    P = d.shape[0]
    NP = x.shape[1]
    N = ss.shape[0]
    M = A.shape[0]

    ssb = _ssb_matmul(d, x, meanY)

    idx2 = patches.reshape(P * NP // _ROW, _ROW)
    val2 = ssb.reshape(P * NP // _ROW, _ROW)
    partials = _sc_scatter(idx2, val2, ss.reshape(N))
    partials = partials.reshape(_NC, N)

    loss = _matvec_mse(
        partials,
        ds.reshape(1, N),
        npp.reshape(1, N),
        vb.reshape(1, N),
        sRef.reshape(1, N),
        A,
        Tarr,
        lam2.reshape(1, 1),
    )
    return loss.reshape(())
